# baseline (speedup 1.0000x reference)
import dataclasses
import functools

import jax
import jax.numpy as jnp
import numpy as np
from jax import lax
from jax.experimental import pallas as pl
from jax.experimental.pallas import tpu as pltpu
from jax.experimental.pallas import tpu_sc as plsc

N = 10000
E = 320000
IN_DIM = 128
HID = 128
NLAYER = 3
NGRAPH = 64
D = HID * NLAYER

NC = 2
NS = 16
NW = NC * NS
CHUNK = 64
NCH = 160
NCH0 = 80
NCH1 = 2 * NCH - NCH0
EP = NW * NCH * CHUNK
IB = 16
RPS = 624


def _sc_aggr_body(h_hbm, src_hbm, dst_hbm, w_hbm, out_hbm,
                  srcs_v, dsts_v, w_v,
                  rows0, rows1, ob0, ob1,
                  acc_sh,
                  gs0, gs1, ss0, ss1):
    i32 = jnp.int32
    cid = lax.axis_index("c").astype(i32)
    sid = lax.axis_index("s").astype(i32)
    wid = cid * i32(NS) + sid
    rows = [rows0, rows1]
    obs = [ob0, ob1]
    gsems = [gs0, gs1]
    ssems = [ss0, ss1]

    lane = lax.iota(jnp.int32, 16)
    col_idx = [lane + i32(g * 16) for g in range(HID // 16)]
    zero16 = jnp.zeros((16,), jnp.float32)

    @pl.loop(jnp.int32(0), jnp.int32(CHUNK), step=jnp.int32(1))
    def _(r):
        row_idx = jnp.full((16,), r.astype(i32), jnp.int32)
        for g in range(HID // 16):
            plsc.store_scatter(rows0, [row_idx, col_idx[g]], zero16)
            plsc.store_scatter(ob0, [row_idx, col_idx[g]], zero16)
            plsc.store_scatter(ob1, [row_idx, col_idx[g]], zero16)

    @pl.loop(jnp.int32(0), jnp.int32(10), step=jnp.int32(1))
    def _(j):
        start = jnp.minimum(sid * i32(RPS) + j.astype(i32) * i32(CHUNK),
                            i32(N - CHUNK))
        pltpu.sync_copy(rows0, acc_sh.at[pl.ds(start, CHUNK)])

    nch = jnp.where(cid == i32(0), i32(NCH0), i32(NCH1))
    cbase = jnp.where(cid == i32(0), sid * i32(NCH0),
                      i32(NS * NCH0) + sid * i32(NCH1))

    pltpu.sync_copy(dst_hbm.at[pl.ds(cbase, IB)], dsts_v)
    for ob in range(2):
        pltpu.async_copy(obs[ob], acc_sh.at[dsts_v.at[i32(0)]], ssems[ob],
                         add=True)

    plsc.subcore_barrier()

    @pl.loop(jnp.int32(0), nch // i32(IB), step=jnp.int32(1))
    def _(blk):
        row0b = cbase + blk.astype(i32) * i32(IB)
        pltpu.sync_copy(src_hbm.at[pl.ds(row0b, IB)], srcs_v)
        pltpu.sync_copy(dst_hbm.at[pl.ds(row0b, IB)], dsts_v)
        pltpu.sync_copy(w_hbm.at[pl.ds(row0b, IB)], w_v)

        for b in range(2):
            pltpu.async_copy(h_hbm.at[srcs_v.at[i32(b)]], rows[b], gsems[b])

        @pl.loop(jnp.int32(0), jnp.int32(IB // 2), step=jnp.int32(1))
        def _(p):
            for b in range(2):
                cl = p.astype(i32) * i32(2) + i32(b)
                clidx = jnp.full((16,), cl, jnp.int32)
                pltpu.make_async_copy(h_hbm.at[pl.ds(0, CHUNK)], rows[b],
                                      gsems[b]).wait()
                pltpu.make_async_copy(h_hbm.at[pl.ds(0, CHUNK)], obs[b],
                                      ssems[b]).wait()

                @plsc.parallel_loop(jnp.int32(0), jnp.int32(CHUNK),
                                    step=jnp.int32(1), unroll=4)
                def _(e):
                    row_idx = jnp.full((16,), e.astype(i32), jnp.int32)
                    wb = plsc.load_gather(w_v, [clidx, row_idx])
                    for g in range(HID // 16):
                        v = plsc.load_gather(rows[b], [row_idx, col_idx[g]])
                        plsc.store_scatter(obs[b], [row_idx, col_idx[g]],
                                           v * wb)

                cn = jnp.minimum(cl + i32(2), i32(IB - 1))
                pltpu.async_copy(h_hbm.at[srcs_v.at[cn]], rows[b], gsems[b])
                pltpu.async_copy(obs[b], acc_sh.at[dsts_v.at[cl]], ssems[b],
                                 add=True)

        for b in range(2):
            pltpu.make_async_copy(h_hbm.at[pl.ds(0, CHUNK)], rows[b],
                                  gsems[b]).wait()

    for b in range(2):
        pltpu.make_async_copy(h_hbm.at[pl.ds(0, CHUNK)], obs[b],
                              ssems[b]).wait()

    plsc.subcore_barrier()

    @pl.when(sid != i32(NS - 1))
    def _():
        pltpu.sync_copy(acc_sh.at[pl.ds(sid * i32(RPS), RPS)],
                        out_hbm.at[cid, pl.ds(sid * i32(RPS), RPS)])

    @pl.when(sid == i32(NS - 1))
    def _():
        last = (NS - 1) * RPS
        pltpu.sync_copy(acc_sh.at[pl.ds(i32(last), N - last)],
                        out_hbm.at[cid, pl.ds(i32(last), N - last)])


def _sc_aggr(h, src_p, dst_p, w_p):
    mesh = plsc.VectorSubcoreMesh(core_axis_name="c", subcore_axis_name="s")
    cp = pltpu.CompilerParams()
    if "needs_layout_passes" in pltpu.CompilerParams.__dataclass_fields__:
        cp = dataclasses.replace(cp, needs_layout_passes=False)
    kern = pl.kernel(
        _sc_aggr_body,
        out_type=jax.ShapeDtypeStruct((NC, N, HID), jnp.float32),
        mesh=mesh,
        compiler_params=cp,
        scratch_types=[
            pltpu.VMEM((IB, CHUNK), jnp.int32),
            pltpu.VMEM((IB, CHUNK), jnp.int32),
            pltpu.VMEM((IB, CHUNK), jnp.float32),
            pltpu.VMEM((CHUNK, HID), jnp.float32),
            pltpu.VMEM((CHUNK, HID), jnp.float32),
            pltpu.VMEM((CHUNK, HID), jnp.float32),
            pltpu.VMEM((CHUNK, HID), jnp.float32),
            pltpu.VMEM_SHARED((N, HID), jnp.float32),
            pltpu.SemaphoreType.DMA,
            pltpu.SemaphoreType.DMA,
            pltpu.SemaphoreType.DMA,
            pltpu.SemaphoreType.DMA,
        ],
    )
    return kern(h, src_p, dst_p, w_p)


_Z = np.int32(0)
RB = 1000
GRID = N // RB


def _ln(z, g, b):
    m = jnp.mean(z, axis=-1, keepdims=True)
    v = jnp.mean((z - m) ** 2, axis=-1, keepdims=True)
    return (z - m) * jax.lax.rsqrt(v + 1e-5) * g + b


def _lin_body(x_ref, w_ref, b_ref, o_ref):
    o_ref[...] = jnp.dot(x_ref[...], w_ref[...],
                         preferred_element_type=jnp.float32) + b_ref[...]


def _tc_lin(x, wT, b):
    return pl.pallas_call(
        _lin_body,
        grid=(GRID,),
        in_specs=[
            pl.BlockSpec((RB, IN_DIM), lambda i: (i, _Z)),
            pl.BlockSpec((IN_DIM, HID), lambda i: (_Z, _Z)),
            pl.BlockSpec((1, HID), lambda i: (_Z, _Z)),
        ],
        out_specs=pl.BlockSpec((RB, HID), lambda i: (i, _Z)),
        out_shape=jax.ShapeDtypeStruct((N, HID), jnp.float32),
        compiler_params=pltpu.CompilerParams(
            dimension_semantics=("parallel",)),
    )(x, wT, b)


def _gin_body(a_ref, w1_ref, b1_ref, g1_ref, be1_ref, w2_ref, b2_ref,
              ng_ref, nb_ref, o_ref):
    aggr = a_ref[0] + a_ref[1]
    z = jnp.dot(aggr, w1_ref[...], preferred_element_type=jnp.float32) \
        + b1_ref[...]
    z = jnp.maximum(_ln(z, g1_ref[...], be1_ref[...]), 0.0)
    h = jnp.dot(z, w2_ref[...], preferred_element_type=jnp.float32) \
        + b2_ref[...] + aggr
    h = jnp.maximum(_ln(h, ng_ref[...], nb_ref[...]), 0.0)
    o_ref[...] = h


def _tc_gin(aggr2, w1T, b1, g1, be1, w2T, b2, ng, nb):
    return pl.pallas_call(
        _gin_body,
        grid=(GRID,),
        in_specs=[
            pl.BlockSpec((NC, RB, HID), lambda i: (_Z, i, _Z)),
            pl.BlockSpec((HID, 2 * HID), lambda i: (_Z, _Z)),
            pl.BlockSpec((1, 2 * HID), lambda i: (_Z, _Z)),
            pl.BlockSpec((1, 2 * HID), lambda i: (_Z, _Z)),
            pl.BlockSpec((1, 2 * HID), lambda i: (_Z, _Z)),
            pl.BlockSpec((2 * HID, HID), lambda i: (_Z, _Z)),
            pl.BlockSpec((1, HID), lambda i: (_Z, _Z)),
            pl.BlockSpec((1, HID), lambda i: (_Z, _Z)),
            pl.BlockSpec((1, HID), lambda i: (_Z, _Z)),
        ],
        out_specs=pl.BlockSpec((RB, HID), lambda i: (i, _Z)),
        out_shape=jax.ShapeDtypeStruct((N, HID), jnp.float32),
        compiler_params=pltpu.CompilerParams(
            dimension_semantics=("parallel",)),
    )(aggr2, w1T, b1, g1, be1, w2T, b2, ng, nb)


def _head_ffn(x, w1_ref, b1_ref, w2_ref, b2_ref, w3_ref, b3_ref,
              ws_ref, bs_ref):
    t = jnp.maximum(jnp.dot(x, w1_ref[...],
                            preferred_element_type=jnp.float32) + b1_ref[...],
                    0.0)
    t = jnp.maximum(jnp.dot(t, w2_ref[...],
                            preferred_element_type=jnp.float32) + b2_ref[...],
                    0.0)
    t = jnp.maximum(jnp.dot(t, w3_ref[...],
                            preferred_element_type=jnp.float32) + b3_ref[...],
                    0.0)
    return t + jnp.dot(x, ws_ref[...],
                       preferred_element_type=jnp.float32) + bs_ref[...]


def _nhead_body(h_ref, batch_ref, mask_ref,
                w1_ref, b1_ref, w2_ref, b2_ref, w3_ref, b3_ref,
                ws_ref, bs_ref, hh_ref, pool_ref):
    hh = _head_ffn(h_ref[...], w1_ref, b1_ref, w2_ref, b2_ref,
                   w3_ref, b3_ref, ws_ref, bs_ref)
    hh_ref[...] = hh
    i = pl.program_id(0)
    hm = hh * mask_ref[0, 0].reshape(RB, 1)
    seg = batch_ref[0, 0].reshape(1, RB)
    onehotT = (seg == lax.broadcasted_iota(jnp.int32, (NGRAPH, RB), 0)
               ).astype(jnp.float32)
    part = jnp.dot(onehotT, hm, preferred_element_type=jnp.float32)

    @pl.when(i == 0)
    def _():
        pool_ref[...] = jnp.zeros_like(pool_ref)

    pool_ref[...] += part


def _tc_nhead(h, batch2d, maskf2d, w1T, b1, w2T, b2, w3T, b3, wsT, bs):
    return pl.pallas_call(
        _nhead_body,
        grid=(GRID,),
        in_specs=[
            pl.BlockSpec((RB, HID), lambda i: (i, _Z)),
            pl.BlockSpec((1, 1, RB), lambda i: (i, _Z, _Z)),
            pl.BlockSpec((1, 1, RB), lambda i: (i, _Z, _Z)),
            pl.BlockSpec((HID, 2 * HID), lambda i: (_Z, _Z)),
            pl.BlockSpec((1, 2 * HID), lambda i: (_Z, _Z)),
            pl.BlockSpec((2 * HID, 2 * HID), lambda i: (_Z, _Z)),
            pl.BlockSpec((1, 2 * HID), lambda i: (_Z, _Z)),
            pl.BlockSpec((2 * HID, HID), lambda i: (_Z, _Z)),
            pl.BlockSpec((1, HID), lambda i: (_Z, _Z)),
            pl.BlockSpec((HID, HID), lambda i: (_Z, _Z)),
            pl.BlockSpec((1, HID), lambda i: (_Z, _Z)),
        ],
        out_specs=[
            pl.BlockSpec((RB, HID), lambda i: (i, _Z)),
            pl.BlockSpec((NGRAPH, HID), lambda i: (_Z, _Z)),
        ],
        out_shape=[
            jax.ShapeDtypeStruct((N, HID), jnp.float32),
            jax.ShapeDtypeStruct((NGRAPH, HID), jnp.float32),
        ],
        compiler_params=pltpu.CompilerParams(
            dimension_semantics=("arbitrary",)),
    )(h, batch2d, maskf2d, w1T, b1, w2T, b2, w3T, b3, wsT, bs)


def _ghead_body(g_ref, w1_ref, b1_ref, w2_ref, b2_ref, w3_ref, b3_ref,
                ws_ref, bs_ref, o_ref):
    o_ref[...] = _head_ffn(g_ref[...], w1_ref, b1_ref, w2_ref, b2_ref,
                           w3_ref, b3_ref, ws_ref, bs_ref)


def _tc_ghead(g, w1T, b1, w2T, b2, w3T, b3, wsT, bs):
    return pl.pallas_call(
        _ghead_body,
        out_shape=jax.ShapeDtypeStruct((NGRAPH, D), jnp.float32),
    )(g, w1T, b1, w2T, b2, w3T, b3, wsT, bs)


def kernel(x, edge_index, edge_weight, batch, mask, params):
    src = edge_index[0].astype(jnp.int32)
    dst = edge_index[1].astype(jnp.int32)
    pad = EP - E
    src_p = jnp.concatenate(
        [src, jnp.zeros((pad,), jnp.int32)]).reshape(EP // CHUNK, CHUNK)
    dst_p = jnp.concatenate(
        [dst, jnp.zeros((pad,), jnp.int32)]).reshape(EP // CHUNK, CHUNK)
    w_p = jnp.concatenate(
        [edge_weight, jnp.zeros((pad,), jnp.float32)]).reshape(
            EP // CHUNK, CHUNK)
    batch2d = batch.astype(jnp.int32).reshape(GRID, 1, RB)
    maskf2d = mask.astype(jnp.float32).reshape(GRID, 1, RB)
    p = params

    def row2d(b):
        return b.reshape(1, -1)

    h = _tc_lin(x, p['lin_w'].T, row2d(p['lin_b']))

    hs = []
    pools = []
    for l in range(NLAYER):
        aggr2 = _sc_aggr(h, src_p, dst_p, w_p)
        h = _tc_gin(aggr2,
                    p['gin_w1'][l].T, row2d(p['gin_b1'][l]),
                    row2d(p['gin_g'][l]), row2d(p['gin_be'][l]),
                    p['gin_w2'][l].T, row2d(p['gin_b2'][l]),
                    row2d(p['norm_g'][l]), row2d(p['norm_b'][l]))
        hh, pool = _tc_nhead(h, batch2d, maskf2d,
                             p['nh_w1'].T, row2d(p['nh_b1']),
                             p['nh_w2'].T, row2d(p['nh_b2']),
                             p['nh_w3'].T, row2d(p['nh_b3']),
                             p['nh_ws'].T, row2d(p['nh_bs']))
        hs.append(hh)
        pools.append(pool)

    g_in = jnp.concatenate(pools, axis=1)
    g = _tc_ghead(g_in,
                  p['gh_w1'].T, row2d(p['gh_b1']),
                  p['gh_w2'].T, row2d(p['gh_b2']),
                  p['gh_w3'].T, row2d(p['gh_b3']),
                  p['gh_ws'].T, row2d(p['gh_bs']))
    return (jnp.concatenate(hs, axis=1), g)


# speedup vs baseline: 1.1964x; 1.1964x over previous
import dataclasses
import functools

import jax
import jax.numpy as jnp
import numpy as np
from jax import lax
from jax.experimental import pallas as pl
from jax.experimental.pallas import tpu as pltpu
from jax.experimental.pallas import tpu_sc as plsc

N = 10000
E = 320000
IN_DIM = 128
HID = 128
NLAYER = 3
NGRAPH = 64
D = HID * NLAYER

NC = 2
NS = 16
NW = NC * NS
CHUNK = 64
NCH = 160
NCH0 = 240
NCH1 = 2 * NCH - NCH0
EP = NW * NCH * CHUNK
IB = 16
RPS = 624


def _sc_aggr_body(h_hbm, src_hbm, dst_hbm, w_hbm, out_hbm,
                  srcs_v, dsts_v, w_v,
                  rows0, rows1, ob0, ob1,
                  acc_sh,
                  gs0, gs1, ss0, ss1):
    i32 = jnp.int32
    cid = lax.axis_index("c").astype(i32)
    sid = lax.axis_index("s").astype(i32)
    wid = cid * i32(NS) + sid
    rows = [rows0, rows1]
    obs = [ob0, ob1]
    gsems = [gs0, gs1]
    ssems = [ss0, ss1]

    lane = lax.iota(jnp.int32, 16)
    col_idx = [lane + i32(g * 16) for g in range(HID // 16)]
    zero16 = jnp.zeros((16,), jnp.float32)

    @pl.loop(jnp.int32(0), jnp.int32(CHUNK), step=jnp.int32(1))
    def _(r):
        row_idx = jnp.full((16,), r.astype(i32), jnp.int32)
        for g in range(HID // 16):
            plsc.store_scatter(rows0, [row_idx, col_idx[g]], zero16)
            plsc.store_scatter(ob0, [row_idx, col_idx[g]], zero16)
            plsc.store_scatter(ob1, [row_idx, col_idx[g]], zero16)

    @pl.loop(jnp.int32(0), jnp.int32(10), step=jnp.int32(1))
    def _(j):
        start = jnp.minimum(sid * i32(RPS) + j.astype(i32) * i32(CHUNK),
                            i32(N - CHUNK))
        pltpu.sync_copy(rows0, acc_sh.at[pl.ds(start, CHUNK)])

    nch = jnp.where(cid == i32(0), i32(NCH0), i32(NCH1))
    cbase = jnp.where(cid == i32(0), sid * i32(NCH0),
                      i32(NS * NCH0) + sid * i32(NCH1))

    pltpu.sync_copy(dst_hbm.at[pl.ds(cbase, IB)], dsts_v)
    for ob in range(2):
        pltpu.async_copy(obs[ob], acc_sh.at[dsts_v.at[i32(0)]], ssems[ob],
                         add=True)

    plsc.subcore_barrier()

    @pl.loop(jnp.int32(0), nch // i32(IB), step=jnp.int32(1))
    def _(blk):
        row0b = cbase + blk.astype(i32) * i32(IB)
        pltpu.sync_copy(src_hbm.at[pl.ds(row0b, IB)], srcs_v)
        pltpu.sync_copy(dst_hbm.at[pl.ds(row0b, IB)], dsts_v)
        pltpu.sync_copy(w_hbm.at[pl.ds(row0b, IB)], w_v)

        for b in range(2):
            pltpu.async_copy(h_hbm.at[srcs_v.at[i32(b)]], rows[b], gsems[b])

        @pl.loop(jnp.int32(0), jnp.int32(IB // 2), step=jnp.int32(1))
        def _(p):
            for b in range(2):
                cl = p.astype(i32) * i32(2) + i32(b)
                clidx = jnp.full((16,), cl, jnp.int32)
                pltpu.make_async_copy(h_hbm.at[pl.ds(0, CHUNK)], rows[b],
                                      gsems[b]).wait()
                pltpu.make_async_copy(h_hbm.at[pl.ds(0, CHUNK)], obs[b],
                                      ssems[b]).wait()

                @plsc.parallel_loop(jnp.int32(0), jnp.int32(CHUNK),
                                    step=jnp.int32(1), unroll=4)
                def _(e):
                    row_idx = jnp.full((16,), e.astype(i32), jnp.int32)
                    wb = plsc.load_gather(w_v, [clidx, row_idx])
                    for g in range(HID // 16):
                        v = plsc.load_gather(rows[b], [row_idx, col_idx[g]])
                        plsc.store_scatter(obs[b], [row_idx, col_idx[g]],
                                           v * wb)

                cn = jnp.minimum(cl + i32(2), i32(IB - 1))
                pltpu.async_copy(h_hbm.at[srcs_v.at[cn]], rows[b], gsems[b])
                pltpu.async_copy(obs[b], acc_sh.at[dsts_v.at[cl]], ssems[b],
                                 add=True)

        for b in range(2):
            pltpu.make_async_copy(h_hbm.at[pl.ds(0, CHUNK)], rows[b],
                                  gsems[b]).wait()

    for b in range(2):
        pltpu.make_async_copy(h_hbm.at[pl.ds(0, CHUNK)], obs[b],
                              ssems[b]).wait()

    plsc.subcore_barrier()

    @pl.when(sid != i32(NS - 1))
    def _():
        pltpu.sync_copy(acc_sh.at[pl.ds(sid * i32(RPS), RPS)],
                        out_hbm.at[cid, pl.ds(sid * i32(RPS), RPS)])

    @pl.when(sid == i32(NS - 1))
    def _():
        last = (NS - 1) * RPS
        pltpu.sync_copy(acc_sh.at[pl.ds(i32(last), N - last)],
                        out_hbm.at[cid, pl.ds(i32(last), N - last)])


def _sc_aggr(h, src_p, dst_p, w_p):
    mesh = plsc.VectorSubcoreMesh(core_axis_name="c", subcore_axis_name="s")
    cp = pltpu.CompilerParams()
    if "needs_layout_passes" in pltpu.CompilerParams.__dataclass_fields__:
        cp = dataclasses.replace(cp, needs_layout_passes=False)
    kern = pl.kernel(
        _sc_aggr_body,
        out_type=jax.ShapeDtypeStruct((NC, N, HID), jnp.float32),
        mesh=mesh,
        compiler_params=cp,
        scratch_types=[
            pltpu.VMEM((IB, CHUNK), jnp.int32),
            pltpu.VMEM((IB, CHUNK), jnp.int32),
            pltpu.VMEM((IB, CHUNK), jnp.float32),
            pltpu.VMEM((CHUNK, HID), jnp.float32),
            pltpu.VMEM((CHUNK, HID), jnp.float32),
            pltpu.VMEM((CHUNK, HID), jnp.float32),
            pltpu.VMEM((CHUNK, HID), jnp.float32),
            pltpu.VMEM_SHARED((N, HID), jnp.float32),
            pltpu.SemaphoreType.DMA,
            pltpu.SemaphoreType.DMA,
            pltpu.SemaphoreType.DMA,
            pltpu.SemaphoreType.DMA,
        ],
    )
    return kern(h, src_p, dst_p, w_p)


_Z = np.int32(0)
RB = 1000
GRID = N // RB


def _ln(z, g, b):
    m = jnp.mean(z, axis=-1, keepdims=True)
    v = jnp.mean((z - m) ** 2, axis=-1, keepdims=True)
    return (z - m) * jax.lax.rsqrt(v + 1e-5) * g + b


def _lin_body(x_ref, w_ref, b_ref, o_ref):
    o_ref[...] = jnp.dot(x_ref[...], w_ref[...],
                         preferred_element_type=jnp.float32) + b_ref[...]


def _tc_lin(x, wT, b):
    return pl.pallas_call(
        _lin_body,
        grid=(GRID,),
        in_specs=[
            pl.BlockSpec((RB, IN_DIM), lambda i: (i, _Z)),
            pl.BlockSpec((IN_DIM, HID), lambda i: (_Z, _Z)),
            pl.BlockSpec((1, HID), lambda i: (_Z, _Z)),
        ],
        out_specs=pl.BlockSpec((RB, HID), lambda i: (i, _Z)),
        out_shape=jax.ShapeDtypeStruct((N, HID), jnp.float32),
        compiler_params=pltpu.CompilerParams(
            dimension_semantics=("parallel",)),
    )(x, wT, b)


def _gin_body(a_ref, w1_ref, b1_ref, g1_ref, be1_ref, w2_ref, b2_ref,
              ng_ref, nb_ref, o_ref):
    aggr = a_ref[0] + a_ref[1]
    z = jnp.dot(aggr, w1_ref[...], preferred_element_type=jnp.float32) \
        + b1_ref[...]
    z = jnp.maximum(_ln(z, g1_ref[...], be1_ref[...]), 0.0)
    h = jnp.dot(z, w2_ref[...], preferred_element_type=jnp.float32) \
        + b2_ref[...] + aggr
    h = jnp.maximum(_ln(h, ng_ref[...], nb_ref[...]), 0.0)
    o_ref[...] = h


def _tc_gin(aggr2, w1T, b1, g1, be1, w2T, b2, ng, nb):
    return pl.pallas_call(
        _gin_body,
        grid=(GRID,),
        in_specs=[
            pl.BlockSpec((NC, RB, HID), lambda i: (_Z, i, _Z)),
            pl.BlockSpec((HID, 2 * HID), lambda i: (_Z, _Z)),
            pl.BlockSpec((1, 2 * HID), lambda i: (_Z, _Z)),
            pl.BlockSpec((1, 2 * HID), lambda i: (_Z, _Z)),
            pl.BlockSpec((1, 2 * HID), lambda i: (_Z, _Z)),
            pl.BlockSpec((2 * HID, HID), lambda i: (_Z, _Z)),
            pl.BlockSpec((1, HID), lambda i: (_Z, _Z)),
            pl.BlockSpec((1, HID), lambda i: (_Z, _Z)),
            pl.BlockSpec((1, HID), lambda i: (_Z, _Z)),
        ],
        out_specs=pl.BlockSpec((RB, HID), lambda i: (i, _Z)),
        out_shape=jax.ShapeDtypeStruct((N, HID), jnp.float32),
        compiler_params=pltpu.CompilerParams(
            dimension_semantics=("parallel",)),
    )(aggr2, w1T, b1, g1, be1, w2T, b2, ng, nb)


def _head_ffn(x, w1_ref, b1_ref, w2_ref, b2_ref, w3_ref, b3_ref,
              ws_ref, bs_ref):
    t = jnp.maximum(jnp.dot(x, w1_ref[...],
                            preferred_element_type=jnp.float32) + b1_ref[...],
                    0.0)
    t = jnp.maximum(jnp.dot(t, w2_ref[...],
                            preferred_element_type=jnp.float32) + b2_ref[...],
                    0.0)
    t = jnp.maximum(jnp.dot(t, w3_ref[...],
                            preferred_element_type=jnp.float32) + b3_ref[...],
                    0.0)
    return t + jnp.dot(x, ws_ref[...],
                       preferred_element_type=jnp.float32) + bs_ref[...]


def _nhead_body(h_ref, batch_ref, mask_ref,
                w1_ref, b1_ref, w2_ref, b2_ref, w3_ref, b3_ref,
                ws_ref, bs_ref, hh_ref, pool_ref):
    hh = _head_ffn(h_ref[...], w1_ref, b1_ref, w2_ref, b2_ref,
                   w3_ref, b3_ref, ws_ref, bs_ref)
    hh_ref[...] = hh
    i = pl.program_id(0)
    hm = hh * mask_ref[0, 0].reshape(RB, 1)
    seg = batch_ref[0, 0].reshape(1, RB)
    onehotT = (seg == lax.broadcasted_iota(jnp.int32, (NGRAPH, RB), 0)
               ).astype(jnp.float32)
    part = jnp.dot(onehotT, hm, preferred_element_type=jnp.float32)

    @pl.when(i == 0)
    def _():
        pool_ref[...] = jnp.zeros_like(pool_ref)

    pool_ref[...] += part


def _tc_nhead(h, batch2d, maskf2d, w1T, b1, w2T, b2, w3T, b3, wsT, bs):
    return pl.pallas_call(
        _nhead_body,
        grid=(GRID,),
        in_specs=[
            pl.BlockSpec((RB, HID), lambda i: (i, _Z)),
            pl.BlockSpec((1, 1, RB), lambda i: (i, _Z, _Z)),
            pl.BlockSpec((1, 1, RB), lambda i: (i, _Z, _Z)),
            pl.BlockSpec((HID, 2 * HID), lambda i: (_Z, _Z)),
            pl.BlockSpec((1, 2 * HID), lambda i: (_Z, _Z)),
            pl.BlockSpec((2 * HID, 2 * HID), lambda i: (_Z, _Z)),
            pl.BlockSpec((1, 2 * HID), lambda i: (_Z, _Z)),
            pl.BlockSpec((2 * HID, HID), lambda i: (_Z, _Z)),
            pl.BlockSpec((1, HID), lambda i: (_Z, _Z)),
            pl.BlockSpec((HID, HID), lambda i: (_Z, _Z)),
            pl.BlockSpec((1, HID), lambda i: (_Z, _Z)),
        ],
        out_specs=[
            pl.BlockSpec((RB, HID), lambda i: (i, _Z)),
            pl.BlockSpec((NGRAPH, HID), lambda i: (_Z, _Z)),
        ],
        out_shape=[
            jax.ShapeDtypeStruct((N, HID), jnp.float32),
            jax.ShapeDtypeStruct((NGRAPH, HID), jnp.float32),
        ],
        compiler_params=pltpu.CompilerParams(
            dimension_semantics=("arbitrary",)),
    )(h, batch2d, maskf2d, w1T, b1, w2T, b2, w3T, b3, wsT, bs)


def _ghead_body(g_ref, w1_ref, b1_ref, w2_ref, b2_ref, w3_ref, b3_ref,
                ws_ref, bs_ref, o_ref):
    o_ref[...] = _head_ffn(g_ref[...], w1_ref, b1_ref, w2_ref, b2_ref,
                           w3_ref, b3_ref, ws_ref, bs_ref)


def _tc_ghead(g, w1T, b1, w2T, b2, w3T, b3, wsT, bs):
    return pl.pallas_call(
        _ghead_body,
        out_shape=jax.ShapeDtypeStruct((NGRAPH, D), jnp.float32),
    )(g, w1T, b1, w2T, b2, w3T, b3, wsT, bs)


def kernel(x, edge_index, edge_weight, batch, mask, params):
    src = edge_index[0].astype(jnp.int32)
    dst = edge_index[1].astype(jnp.int32)
    pad = EP - E
    src_p = jnp.concatenate(
        [src, jnp.zeros((pad,), jnp.int32)]).reshape(EP // CHUNK, CHUNK)
    dst_p = jnp.concatenate(
        [dst, jnp.zeros((pad,), jnp.int32)]).reshape(EP // CHUNK, CHUNK)
    w_p = jnp.concatenate(
        [edge_weight, jnp.zeros((pad,), jnp.float32)]).reshape(
            EP // CHUNK, CHUNK)
    batch2d = batch.astype(jnp.int32).reshape(GRID, 1, RB)
    maskf2d = mask.astype(jnp.float32).reshape(GRID, 1, RB)
    p = params

    def row2d(b):
        return b.reshape(1, -1)

    h = _tc_lin(x, p['lin_w'].T, row2d(p['lin_b']))

    hs = []
    pools = []
    for l in range(NLAYER):
        aggr2 = _sc_aggr(h, src_p, dst_p, w_p)
        h = _tc_gin(aggr2,
                    p['gin_w1'][l].T, row2d(p['gin_b1'][l]),
                    row2d(p['gin_g'][l]), row2d(p['gin_be'][l]),
                    p['gin_w2'][l].T, row2d(p['gin_b2'][l]),
                    row2d(p['norm_g'][l]), row2d(p['norm_b'][l]))
        hh, pool = _tc_nhead(h, batch2d, maskf2d,
                             p['nh_w1'].T, row2d(p['nh_b1']),
                             p['nh_w2'].T, row2d(p['nh_b2']),
                             p['nh_w3'].T, row2d(p['nh_b3']),
                             p['nh_ws'].T, row2d(p['nh_bs']))
        hs.append(hh)
        pools.append(pool)

    g_in = jnp.concatenate(pools, axis=1)
    g = _tc_ghead(g_in,
                  p['gh_w1'].T, row2d(p['gh_b1']),
                  p['gh_w2'].T, row2d(p['gh_b2']),
                  p['gh_w3'].T, row2d(p['gh_b3']),
                  p['gh_ws'].T, row2d(p['gh_bs']))
    return (jnp.concatenate(hs, axis=1), g)
